# Initial kernel scaffold; baseline (speedup 1.0000x reference)
import functools

import jax
import jax.numpy as jnp
from jax import lax
from jax.experimental import pallas as pl
from jax.experimental.pallas import tpu as pltpu
from jax.experimental.pallas import tpu_sc as plsc

NN = 100000
EE = 1600000
FIN = 128
HH = 32
HHH = 16
OUTD = 16
GG = 64

NC = 2
NS = 16
ROW = 128
RPC = 8
CHUNK = ROW * RPC

E_ROWS = 12544
E_PAD = E_ROWS * ROW
ROWS_PER_TILE = E_ROWS // NS
CHUNKS_PER_TILE = ROWS_PER_TILE // RPC

NACC = 100016
NP = 100096
DEG_ROWS_PER_W = E_ROWS // (NC * NS)

BLK = 4000

_mesh = plsc.VectorSubcoreMesh(
    core_axis_name="c", subcore_axis_name="s", num_cores=NC, num_subcores=NS)


@functools.partial(
    pl.kernel,
    out_type=jax.ShapeDtypeStruct((2 * NP,), jnp.float32),
    mesh=_mesh,
    scratch_types=[
        pltpu.VMEM_SHARED((NP,), jnp.float32),
        pltpu.VMEM((RPC, ROW), jnp.int32),
        pltpu.VMEM((ROW,), jnp.float32),
    ],
)
def _deg_kernel(dst_hbm, zeros_hbm, out_hbm, acc, dbuf, ones_v):
    c = lax.axis_index("c")
    s = lax.axis_index("s")
    w = c * NS + s
    per = NP // NS
    pltpu.sync_copy(zeros_hbm, acc.at[pl.ds(s * per, per)])
    for i in range(RPC):
        ones_v[pl.ds(i * 16, 16)] = jnp.ones((16,), jnp.float32)
    plsc.subcore_barrier()

    def body(i, carry):
        row0 = w * DEG_ROWS_PER_W + i * RPC
        pltpu.sync_copy(dst_hbm.at[pl.ds(row0, RPC)], dbuf)
        for j in range(RPC):
            pltpu.sync_copy(ones_v, acc.at[dbuf.at[j]], add=True)
        return carry

    lax.fori_loop(0, DEG_ROWS_PER_W // RPC, body, 0)
    plsc.subcore_barrier()
    pltpu.sync_copy(acc.at[pl.ds(s * per, per)],
                    out_hbm.at[pl.ds(c * NP + s * per, per)])


@functools.partial(
    pl.kernel,
    out_type=jax.ShapeDtypeStruct((2 * NN, HHH), jnp.float32),
    mesh=_mesh,
    scratch_types=[
        pltpu.VMEM_SHARED((NACC, HHH), jnp.float32),
        pltpu.VMEM((RPC, ROW), jnp.int32),
        pltpu.VMEM((RPC, ROW), jnp.int32),
        pltpu.VMEM((CHUNK, HHH), jnp.float32),
        pltpu.SemaphoreType.DMA,
    ],
)
def _scatter_kernel(g_hbm, src_hbm, dst_hbm, out_hbm, acc, sbuf, dbuf, rows,
                    gsem):
    c = lax.axis_index("c")
    s = lax.axis_index("s")
    per = NN // NS
    pltpu.sync_copy(g_hbm.at[pl.ds(c * NN + s * per, per)],
                    acc.at[pl.ds(s * per, per)])

    @pl.when(s == 0)
    def _():
        pltpu.sync_copy(g_hbm.at[pl.ds(c * NN, NACC - NN)],
                        acc.at[pl.ds(NN, NACC - NN)])

    plsc.subcore_barrier()

    def body(i, carry):
        row0 = s * ROWS_PER_TILE + i * RPC
        pltpu.sync_copy(src_hbm.at[pl.ds(c * E_ROWS + row0, RPC)], sbuf)
        pltpu.sync_copy(dst_hbm.at[pl.ds(row0, RPC)], dbuf)
        cps = [
            pltpu.async_copy(g_hbm.at[sbuf.at[j]],
                             rows.at[pl.ds(j * ROW, ROW)], gsem)
            for j in range(RPC)
        ]
        for j in range(RPC):
            cps[j].wait()
            pltpu.sync_copy(rows.at[pl.ds(j * ROW, ROW)],
                            acc.at[dbuf.at[j]], add=True)
        return carry

    lax.fori_loop(0, CHUNKS_PER_TILE, body, 0)
    plsc.subcore_barrier()
    pltpu.sync_copy(acc.at[pl.ds(s * per, per)],
                    out_hbm.at[pl.ds(c * NN + s * per, per)])


def _tc0_body(x_ref, cnt_ref, w_ref, g_ref, dinv_ref):
    deg = cnt_ref[0] + cnt_ref[1] + 1.0
    dinv = lax.rsqrt(deg)
    dinv_ref[...] = dinv
    h = jnp.dot(x_ref[...], w_ref[...], preferred_element_type=jnp.float32)
    g = h * dinv
    g_ref[0] = g[:, :HHH]
    g_ref[1] = g[:, HHH:]


def _tc0(x, cnt3, w0):
    return pl.pallas_call(
        _tc0_body,
        grid=(NN // BLK,),
        in_specs=[
            pl.BlockSpec((BLK, FIN), lambda i: (i, 0)),
            pl.BlockSpec((2, BLK, 1), lambda i: (0, i, 0)),
            pl.BlockSpec((FIN, HH), lambda i: (0, 0)),
        ],
        out_specs=[
            pl.BlockSpec((2, BLK, HHH), lambda i: (0, i, 0)),
            pl.BlockSpec((BLK, 1), lambda i: (i, 0)),
        ],
        out_shape=[
            jax.ShapeDtypeStruct((2, NN, HHH), jnp.float32),
            jax.ShapeDtypeStruct((NN, 1), jnp.float32),
        ],
    )(x, cnt3, w0)


def _tcmid_body(agg_ref, dinv_ref, b_ref, w_ref, g_ref):
    dinv = dinv_ref[...]
    ssum = jnp.concatenate([agg_ref[0], agg_ref[1]], axis=1)
    o = ssum * dinv + b_ref[...]
    r = jnp.maximum(o, 0.0)
    h = jnp.dot(r, w_ref[...], preferred_element_type=jnp.float32)
    g = h * dinv
    g_ref[0] = g[:, :HHH]
    g_ref[1] = g[:, HHH:]


def _tcmid(agg3, dinv, b, w):
    return pl.pallas_call(
        _tcmid_body,
        grid=(NN // BLK,),
        in_specs=[
            pl.BlockSpec((2, BLK, HHH), lambda i: (0, i, 0)),
            pl.BlockSpec((BLK, 1), lambda i: (i, 0)),
            pl.BlockSpec((1, HH), lambda i: (0, 0)),
            pl.BlockSpec((HH, HH), lambda i: (0, 0)),
        ],
        out_specs=pl.BlockSpec((2, BLK, HHH), lambda i: (0, i, 0)),
        out_shape=jax.ShapeDtypeStruct((2, NN, HHH), jnp.float32),
    )(agg3, dinv, b, w)


def _tcpool_body(agg_ref, dinv_ref, b_ref, batch_ref, wl_ref, bl_ref,
                 out_ref, acc_ref):
    i = pl.program_id(0)

    @pl.when(i == 0)
    def _():
        acc_ref[...] = jnp.zeros_like(acc_ref)

    ssum = jnp.concatenate([agg_ref[0], agg_ref[1]], axis=1)
    h = ssum * dinv_ref[...] + b_ref[...]
    hext = jnp.concatenate([h, jnp.ones((BLK, 1), jnp.float32)], axis=1)
    ids = lax.broadcasted_iota(jnp.int32, (1, GG), 1)
    oh = (batch_ref[...] == ids).astype(jnp.float32)
    acc_ref[...] += lax.dot_general(
        oh, hext, (((0,), (0,)), ((), ())),
        preferred_element_type=jnp.float32)

    @pl.when(i == NN // BLK - 1)
    def _():
        sums = acc_ref[:, :HH]
        cnt = jnp.maximum(acc_ref[:, HH:HH + 1], 1.0)
        pooled = sums / cnt
        out_ref[...] = jnp.dot(
            pooled, wl_ref[...],
            preferred_element_type=jnp.float32) + bl_ref[...]


def _tcpool(agg3, dinv, b, batch2, wl, bl):
    return pl.pallas_call(
        _tcpool_body,
        grid=(NN // BLK,),
        in_specs=[
            pl.BlockSpec((2, BLK, HHH), lambda i: (0, i, 0)),
            pl.BlockSpec((BLK, 1), lambda i: (i, 0)),
            pl.BlockSpec((1, HH), lambda i: (0, 0)),
            pl.BlockSpec((BLK, 1), lambda i: (i, 0)),
            pl.BlockSpec((HH, OUTD), lambda i: (0, 0)),
            pl.BlockSpec((1, OUTD), lambda i: (0, 0)),
        ],
        out_specs=pl.BlockSpec((GG, OUTD), lambda i: (0, 0)),
        out_shape=jax.ShapeDtypeStruct((GG, OUTD), jnp.float32),
        scratch_shapes=[pltpu.VMEM((GG, HH + 1), jnp.float32)],
    )(agg3, dinv, b, batch2, wl, bl)


def kernel(x, edge_index, batch, W0, b0, W1, b1, W2, b2, Wl, bl):
    src = edge_index[0]
    dst = edge_index[1]
    npad = E_PAD - EE
    srcp = jnp.concatenate([src, jnp.zeros((npad,), jnp.int32)])
    dstp = jnp.concatenate([dst, jnp.full((npad,), NN, jnp.int32)])
    src2 = jnp.stack([srcp, srcp + NN]).reshape(2 * E_ROWS, ROW)
    dst3 = dstp.reshape(E_ROWS, ROW)
    zeros = jnp.zeros((NP // NS,), jnp.float32)

    cnt = _deg_kernel(dst3, zeros)
    cnt3 = cnt.reshape(2, NP, 1)

    g3, dinv = _tc0(x, cnt3, W0)

    agg = _scatter_kernel(g3.reshape(2 * NN, HHH), src2, dst3)
    g3 = _tcmid(agg.reshape(2, NN, HHH), dinv, b0.reshape(1, HH), W1)

    agg = _scatter_kernel(g3.reshape(2 * NN, HHH), src2, dst3)
    g3 = _tcmid(agg.reshape(2, NN, HHH), dinv, b1.reshape(1, HH), W2)

    agg = _scatter_kernel(g3.reshape(2 * NN, HHH), src2, dst3)
    return _tcpool(agg.reshape(2, NN, HHH), dinv, b2.reshape(1, HH),
                   batch.reshape(NN, 1), Wl, bl)


# speedup vs baseline: 23.8568x; 23.8568x over previous
import functools

import jax
import jax.numpy as jnp
from jax import lax
from jax.experimental import pallas as pl
from jax.experimental.pallas import tpu as pltpu
from jax.experimental.pallas import tpu_sc as plsc

NN = 100000
EE = 1600000
FIN = 128
HH = 32
HHH = 16
OUTD = 16
GG = 64

NC = 2
NS = 16
ROW = 128
RPC = 8
CHUNK = ROW * RPC

E_ROWS = 12544
E_PAD = E_ROWS * ROW
ROWS_PER_TILE = E_ROWS // NS
CHUNKS_PER_TILE = ROWS_PER_TILE // RPC

NACC = 100096
NPT = NACC // NS
ZCH = 368
NP = 100352
NPP = NP // NS
DEG_ROWS_PER_W = E_ROWS // (NC * NS)

BLK = 4000

_mesh = plsc.VectorSubcoreMesh(
    core_axis_name="c", subcore_axis_name="s", num_cores=NC, num_subcores=NS)


@functools.partial(
    pl.kernel,
    out_type=jax.ShapeDtypeStruct((2 * NP,), jnp.float32),
    mesh=_mesh,
    compiler_params=pltpu.CompilerParams(use_tc_tiling_on_sc=False),
    scratch_types=[
        pltpu.VMEM_SHARED((NP,), jnp.float32),
        pltpu.VMEM((RPC, ROW), jnp.int32),
        pltpu.VMEM((ROW,), jnp.float32),
        pltpu.VMEM((NPP,), jnp.float32),
    ],
)
def _deg_kernel(dst_hbm, out_hbm, acc, dbuf, ones_v, zbuf):
    c = lax.axis_index("c")
    s = lax.axis_index("s")
    w = c * NS + s

    def zfill(i, carry):
        zbuf[pl.ds(pl.multiple_of(i * 16, 16), 16)] = jnp.zeros(
            (16,), jnp.float32)
        return carry

    lax.fori_loop(0, NPP // 16, zfill, 0)
    pltpu.sync_copy(zbuf, acc.at[pl.ds(s * NPP, NPP)])
    for i in range(RPC):
        ones_v[pl.ds(i * 16, 16)] = jnp.ones((16,), jnp.float32)
    plsc.subcore_barrier()

    def body(i, carry):
        row0 = w * DEG_ROWS_PER_W + i * RPC
        pltpu.sync_copy(dst_hbm.at[pl.ds(row0, RPC)], dbuf)
        for j in range(RPC):
            pltpu.sync_copy(ones_v, acc.at[dbuf.at[j]], add=True)
        return carry

    lax.fori_loop(0, DEG_ROWS_PER_W // RPC, body, 0)
    plsc.subcore_barrier()
    pltpu.sync_copy(acc.at[pl.ds(s * NPP, NPP)], zbuf)
    pltpu.sync_copy(zbuf, out_hbm.at[pl.ds(c * NP + s * NPP, NPP)])


@functools.partial(
    pl.kernel,
    out_type=jax.ShapeDtypeStruct((2 * NACC, HHH), jnp.float32),
    mesh=_mesh,
    compiler_params=pltpu.CompilerParams(use_tc_tiling_on_sc=False),
    scratch_types=[
        pltpu.VMEM_SHARED((NACC, HHH), jnp.float32),
        pltpu.VMEM((RPC, ROW), jnp.int32),
        pltpu.VMEM((RPC, ROW), jnp.int32),
        pltpu.VMEM((CHUNK, HHH), jnp.float32),
        pltpu.VMEM((ZCH, HHH), jnp.float32),
        pltpu.SemaphoreType.DMA,
    ],
)
def _scatter_kernel(g_hbm, src_hbm, dst_hbm, out_hbm, acc, sbuf, dbuf, rows,
                    zbuf, gsem):
    c = lax.axis_index("c")
    s = lax.axis_index("s")

    def zfill(i, carry):
        zbuf[i] = jnp.zeros((HHH,), jnp.float32)
        return carry

    lax.fori_loop(0, ZCH, zfill, 0)
    for k in range(NPT // ZCH):
        pltpu.sync_copy(zbuf, acc.at[pl.ds(s * NPT + k * ZCH, ZCH)])

    plsc.subcore_barrier()

    def body(i, carry):
        row0 = s * ROWS_PER_TILE + i * RPC
        pltpu.sync_copy(src_hbm.at[pl.ds(c * E_ROWS + row0, RPC)], sbuf)
        pltpu.sync_copy(dst_hbm.at[pl.ds(row0, RPC)], dbuf)
        cps = [
            pltpu.async_copy(g_hbm.at[sbuf.at[j]],
                             rows.at[pl.ds(j * ROW, ROW)], gsem)
            for j in range(RPC)
        ]
        for j in range(RPC):
            cps[j].wait()
            pltpu.sync_copy(rows.at[pl.ds(j * ROW, ROW)],
                            acc.at[dbuf.at[j]], add=True)
        return carry

    lax.fori_loop(0, CHUNKS_PER_TILE, body, 0)
    plsc.subcore_barrier()
    for k in range(NPT // ZCH):
        pltpu.sync_copy(acc.at[pl.ds(s * NPT + k * ZCH, ZCH)], zbuf)
        pltpu.sync_copy(zbuf, out_hbm.at[pl.ds(c * NACC + s * NPT + k * ZCH,
                                               ZCH)])


def _tc0_body(x_ref, cnt_ref, w_ref, g_ref, dinv_ref):
    deg = cnt_ref[0] + cnt_ref[1] + 1.0
    dinv = lax.rsqrt(deg)
    dinv_ref[...] = dinv
    h = jnp.dot(x_ref[...], w_ref[...], preferred_element_type=jnp.float32)
    g = h * dinv
    g_ref[0] = g[:, :HHH]
    g_ref[1] = g[:, HHH:]


def _tc0(x, cnt3, w0):
    return pl.pallas_call(
        _tc0_body,
        grid=(NN // BLK,),
        in_specs=[
            pl.BlockSpec((BLK, FIN), lambda i: (i, 0)),
            pl.BlockSpec((2, BLK, 1), lambda i: (0, i, 0)),
            pl.BlockSpec((FIN, HH), lambda i: (0, 0)),
        ],
        out_specs=[
            pl.BlockSpec((2, BLK, HHH), lambda i: (0, i, 0)),
            pl.BlockSpec((BLK, 1), lambda i: (i, 0)),
        ],
        out_shape=[
            jax.ShapeDtypeStruct((2, NN, HHH), jnp.float32),
            jax.ShapeDtypeStruct((NN, 1), jnp.float32),
        ],
    )(x, cnt3, w0)


def _tcmid_body(agg_ref, g_ref, dinv_ref, b_ref, w_ref, gout_ref):
    dinv = dinv_ref[...]
    ssum = jnp.concatenate([agg_ref[0] + g_ref[0], agg_ref[1] + g_ref[1]],
                           axis=1)
    o = ssum * dinv + b_ref[...]
    r = jnp.maximum(o, 0.0)
    h = jnp.dot(r, w_ref[...], preferred_element_type=jnp.float32)
    g = h * dinv
    gout_ref[0] = g[:, :HHH]
    gout_ref[1] = g[:, HHH:]


def _tcmid(agg3, g3, dinv, b, w):
    spec = pl.BlockSpec((2, BLK, HHH), lambda i: (0, i, 0))
    return pl.pallas_call(
        _tcmid_body,
        grid=(NN // BLK,),
        in_specs=[
            spec,
            spec,
            pl.BlockSpec((BLK, 1), lambda i: (i, 0)),
            pl.BlockSpec((1, HH), lambda i: (0, 0)),
            pl.BlockSpec((HH, HH), lambda i: (0, 0)),
        ],
        out_specs=spec,
        out_shape=jax.ShapeDtypeStruct((2, NN, HHH), jnp.float32),
    )(agg3, g3, dinv, b, w)


def _tcpool_body(agg_ref, g_ref, dinv_ref, b_ref, batch_ref, wl_ref, bl_ref,
                 out_ref, acc_ref):
    i = pl.program_id(0)

    @pl.when(i == 0)
    def _():
        acc_ref[...] = jnp.zeros_like(acc_ref)

    ssum = jnp.concatenate([agg_ref[0] + g_ref[0], agg_ref[1] + g_ref[1]],
                           axis=1)
    h = ssum * dinv_ref[...] + b_ref[...]
    hext = jnp.concatenate([h, jnp.ones((BLK, 1), jnp.float32)], axis=1)
    ids = lax.broadcasted_iota(jnp.int32, (1, GG), 1)
    oh = (batch_ref[...] == ids).astype(jnp.float32)
    acc_ref[...] += lax.dot_general(
        oh, hext, (((0,), (0,)), ((), ())),
        preferred_element_type=jnp.float32)

    @pl.when(i == NN // BLK - 1)
    def _():
        sums = acc_ref[:, :HH]
        cnt = jnp.maximum(acc_ref[:, HH:HH + 1], 1.0)
        pooled = sums / cnt
        out_ref[...] = jnp.dot(
            pooled, wl_ref[...],
            preferred_element_type=jnp.float32) + bl_ref[...]


def _tcpool(agg3, g3, dinv, b, batch2, wl, bl):
    spec = pl.BlockSpec((2, BLK, HHH), lambda i: (0, i, 0))
    return pl.pallas_call(
        _tcpool_body,
        grid=(NN // BLK,),
        in_specs=[
            spec,
            spec,
            pl.BlockSpec((BLK, 1), lambda i: (i, 0)),
            pl.BlockSpec((1, HH), lambda i: (0, 0)),
            pl.BlockSpec((BLK, 1), lambda i: (i, 0)),
            pl.BlockSpec((HH, OUTD), lambda i: (0, 0)),
            pl.BlockSpec((1, OUTD), lambda i: (0, 0)),
        ],
        out_specs=pl.BlockSpec((GG, OUTD), lambda i: (0, 0)),
        out_shape=jax.ShapeDtypeStruct((GG, OUTD), jnp.float32),
        scratch_shapes=[pltpu.VMEM((GG, HH + 1), jnp.float32)],
    )(agg3, g3, dinv, b, batch2, wl, bl)


def kernel(x, edge_index, batch, W0, b0, W1, b1, W2, b2, Wl, bl):
    src = edge_index[0]
    dst = edge_index[1]
    npad = E_PAD - EE
    srcp = jnp.concatenate([src, jnp.zeros((npad,), jnp.int32)])
    dstp = jnp.concatenate([dst, jnp.full((npad,), NN, jnp.int32)])
    src2 = jnp.stack([srcp, srcp + NN]).reshape(2 * E_ROWS, ROW)
    dst3 = dstp.reshape(E_ROWS, ROW)

    cnt = _deg_kernel(dst3)
    cnt3 = cnt.reshape(2, NP, 1)

    g3, dinv = _tc0(x, cnt3, W0)

    agg = _scatter_kernel(g3.reshape(2 * NN, HHH), src2, dst3)
    g3 = _tcmid(agg.reshape(2, NACC, HHH), g3, dinv, b0.reshape(1, HH), W1)

    agg = _scatter_kernel(g3.reshape(2 * NN, HHH), src2, dst3)
    g3 = _tcmid(agg.reshape(2, NACC, HHH), g3, dinv, b1.reshape(1, HH), W2)

    agg = _scatter_kernel(g3.reshape(2 * NN, HHH), src2, dst3)
    return _tcpool(agg.reshape(2, NACC, HHH), g3, dinv, b2.reshape(1, HH),
                   batch.reshape(NN, 1), Wl, bl.reshape(1, OUTD))
